# baseline (speedup 1.0000x reference)
import jax
import jax.numpy as jnp
from jax import lax
from jax.experimental import pallas as pl
from jax.experimental.pallas import tpu as pltpu
from jax.experimental.pallas import tpu_sc as plsc

B = 256
N = 100000
G = 5000
HALF_G = G // 2

NC = 2
NS = 16
NW = NC * NS
ROWS_PER_TILE = B // NW
NB = 2000
NUM_BLOCKS = N // NB
L = 16


def _sc_kernel(x_hbm, seg_hbm, w_hbm, out_hbm, x_buf, seg_buf, w_buf, acc):
    wid = lax.axis_index("s") * NC + lax.axis_index("c")
    row0 = wid * ROWS_PER_TILE

    zeros = jnp.zeros((L,), jnp.float32)

    def zero_body(i, carry):
        acc[pl.ds(i * L, L)] = zeros
        return carry

    lax.fori_loop(0, ROWS_PER_TILE * G // L, zero_body, 0)

    def block_body(blk, carry):
        off = blk * NB
        pltpu.sync_copy(seg_hbm.at[pl.ds(off, NB)], seg_buf)
        pltpu.sync_copy(w_hbm.at[pl.ds(off, NB)], w_buf)
        for r in range(ROWS_PER_TILE):
            pltpu.sync_copy(x_hbm.at[pl.ds((row0 + r) * N + off, NB)],
                            x_buf.at[r])

        def chunk_body(j, inner):
            sv = seg_buf[pl.ds(j * L, L)]
            wv = w_buf[pl.ds(j * L, L)]
            wf = jnp.where(sv >= HALF_G, wv, jnp.float32(1.0))
            for r in range(ROWS_PER_TILE):
                xv = x_buf[r, pl.ds(j * L, L)]
                plsc.addupdate_scatter(acc, [sv + (r * G)], xv * wf)
            return inner

        lax.fori_loop(0, NB // L, chunk_body, 0)
        return carry

    lax.fori_loop(0, NUM_BLOCKS, block_body, 0)

    pltpu.sync_copy(acc, out_hbm.at[pl.ds(row0 * G, ROWS_PER_TILE * G)])


def kernel(x, segment_ids, W):
    mesh = plsc.VectorSubcoreMesh(core_axis_name="c", subcore_axis_name="s")
    f = pl.kernel(
        _sc_kernel,
        mesh=mesh,
        compiler_params=pltpu.CompilerParams(needs_layout_passes=False, use_tc_tiling_on_sc=False),
        out_type=jax.ShapeDtypeStruct((B * G,), jnp.float32),
        scratch_types=[
            pltpu.VMEM((ROWS_PER_TILE, NB), jnp.float32),
            pltpu.VMEM((NB,), jnp.int32),
            pltpu.VMEM((NB,), jnp.float32),
            pltpu.VMEM((ROWS_PER_TILE * G,), jnp.float32),
        ],
    )
    return f(x.reshape(-1), segment_ids, W).reshape(B, G)


# speedup vs baseline: 1.4391x; 1.4391x over previous
import jax
import jax.numpy as jnp
from jax import lax
from jax.experimental import pallas as pl
from jax.experimental.pallas import tpu as pltpu
from jax.experimental.pallas import tpu_sc as plsc

B = 256
N = 100000
G = 5000
HALF_G = G // 2

NC = 2
NS = 16
NW = NC * NS
R = B // NW
NB = 2000
NUM_BLOCKS = N // NB
L = 16
H = NB // 2
UNROLL = 4


def _sc_kernel(x_hbm, seg_hbm, w_hbm, out_hbm,
               x_buf, seg_buf, w_buf, acc):
    wid = lax.axis_index("s") * NC + lax.axis_index("c")
    row0 = wid * R

    lane = lax.iota(jnp.int32, L)
    lane_r = lane & (R - 1)
    hi = lane >= R
    hi_off = jnp.where(hi, H, 0).astype(jnp.int32)
    gbase = lane_r * NB + hi_off
    sbase = lane_r * G
    zeros = jnp.zeros((L,), jnp.float32)

    def zero_body(i, carry):
        acc[pl.ds(i * L, L)] = zeros
        return carry

    lax.fori_loop(0, R * G // L, zero_body, 0)

    def block_body(blk, carry):
        off = blk * NB
        pltpu.sync_copy(seg_hbm.at[pl.ds(off, NB)], seg_buf)
        pltpu.sync_copy(w_hbm.at[pl.ds(off, NB)], w_buf)
        for r in range(R):
            pltpu.sync_copy(x_hbm.at[pl.ds((row0 + r) * N + off, NB)],
                            x_buf.at[pl.ds(r * NB, NB)])

        def pair_body(jj, jv):
            for u in range(UNROLL):
                jvu = jv + u
                xv = plsc.load_gather(x_buf, [gbase + jvu])
                sv = plsc.load_gather(seg_buf, [hi_off + jvu])
                wv = plsc.load_gather(w_buf, [hi_off + jvu])
                wfv = jnp.where(sv >= HALF_G, wv, jnp.float32(1.0))
                plsc.addupdate_scatter(acc, [sbase + sv], xv * wfv)
            return jv + UNROLL

        lax.fori_loop(0, H // UNROLL, pair_body, jnp.zeros((L,), jnp.int32))
        return carry

    lax.fori_loop(0, NUM_BLOCKS, block_body, 0)

    pltpu.sync_copy(acc, out_hbm.at[pl.ds(row0 * G, R * G)])


def kernel(x, segment_ids, W):
    mesh = plsc.VectorSubcoreMesh(core_axis_name="c", subcore_axis_name="s")
    f = pl.kernel(
        _sc_kernel,
        mesh=mesh,
        compiler_params=pltpu.CompilerParams(
            needs_layout_passes=False, use_tc_tiling_on_sc=False),
        out_type=jax.ShapeDtypeStruct((B * G,), jnp.float32),
        scratch_types=[
            pltpu.VMEM((R * NB,), jnp.float32),
            pltpu.VMEM((NB,), jnp.int32),
            pltpu.VMEM((NB,), jnp.float32),
            pltpu.VMEM((R * G,), jnp.float32),
        ],
    )
    return f(x.reshape(-1), segment_ids, W).reshape(B, G)


# speedup vs baseline: 1.8485x; 1.2845x over previous
import jax
import jax.numpy as jnp
from jax import lax
from jax.experimental import pallas as pl
from jax.experimental.pallas import tpu as pltpu
from jax.experimental.pallas import tpu_sc as plsc

B = 256
N = 100000
G = 5000
HALF_G = G // 2

NC = 2
NS = 16
NW = NC * NS
R = B // NW
NB = 2000
NUM_BLOCKS = N // NB
L = 16
H = NB // 2
UNROLL = 8


def _sc_kernel(x_hbm, seg_hbm, w_hbm, out_hbm,
               x_buf, seg_buf, w_buf, acc):
    wid = lax.axis_index("s") * NC + lax.axis_index("c")
    row0 = wid * R

    lane = lax.iota(jnp.int32, L)
    lane_r = lane & (R - 1)
    hi = lane >= R
    hi_off = jnp.where(hi, H, 0).astype(jnp.int32)
    gbase = lane_r * NB + hi_off
    sbase = lane_r * G
    zeros = jnp.zeros((L,), jnp.float32)

    @plsc.parallel_loop(0, R * G // L, unroll=8)
    def zero_body(i):
        acc[pl.ds(i * L, L)] = zeros

    def block_body(blk, carry):
        off = blk * NB
        pltpu.sync_copy(seg_hbm.at[pl.ds(off, NB)], seg_buf)
        pltpu.sync_copy(w_hbm.at[pl.ds(off, NB)], w_buf)
        for r in range(R):
            pltpu.sync_copy(x_hbm.at[pl.ds((row0 + r) * N + off, NB)],
                            x_buf.at[pl.ds(r * NB, NB)])

        @plsc.parallel_loop(0, H, unroll=UNROLL)
        def pair_body(j):
            gs = hi_off + j
            xv = plsc.load_gather(x_buf, [gbase + j])
            sv = plsc.load_gather(seg_buf, [gs])
            wv = plsc.load_gather(w_buf, [gs])
            wfv = jnp.where(sv >= HALF_G, wv, jnp.float32(1.0))
            plsc.addupdate_scatter(acc, [sbase + sv], xv * wfv)
        return carry

    lax.fori_loop(0, NUM_BLOCKS, block_body, 0)

    pltpu.sync_copy(acc, out_hbm.at[pl.ds(row0 * G, R * G)])


def kernel(x, segment_ids, W):
    mesh = plsc.VectorSubcoreMesh(core_axis_name="c", subcore_axis_name="s")
    f = pl.kernel(
        _sc_kernel,
        mesh=mesh,
        compiler_params=pltpu.CompilerParams(
            needs_layout_passes=False, use_tc_tiling_on_sc=False),
        out_type=jax.ShapeDtypeStruct((B * G,), jnp.float32),
        scratch_types=[
            pltpu.VMEM((R * NB,), jnp.float32),
            pltpu.VMEM((NB,), jnp.int32),
            pltpu.VMEM((NB,), jnp.float32),
            pltpu.VMEM((R * G,), jnp.float32),
        ],
    )
    return f(x.reshape(-1), segment_ids, W).reshape(B, G)


# speedup vs baseline: 3.4253x; 1.8530x over previous
import jax
import jax.numpy as jnp
from jax import lax
from jax.experimental import pallas as pl
from jax.experimental.pallas import tpu as pltpu
from jax.experimental.pallas import tpu_sc as plsc

B = 256
N = 100000
G = 5000
HALF_G = G // 2

NC = 2
NS = 16
NW = NC * NS
R = B // NW
NB = 2000
NUM_BLOCKS = N // NB
L = 16
H = NB // 2
UNROLL = 8


def _sc_kernel(x_hbm, seg_hbm, w_hbm, out_hbm,
               x_buf, seg_buf, w_buf, acc, sem):
    wid = lax.axis_index("s") * NC + lax.axis_index("c")
    row0 = wid * R

    lane = lax.iota(jnp.int32, L)
    lane_r = lane & (R - 1)
    hi = lane >= R
    hi_off = jnp.where(hi, H, 0).astype(jnp.int32)
    gbase = lane_r * NB + hi_off
    sbase = lane_r * G
    zeros = jnp.zeros((L,), jnp.float32)

    def issue(blk, par):
        off = blk * NB
        pltpu.async_copy(seg_hbm.at[pl.ds(off, NB)],
                         seg_buf.at[pl.ds(par * NB, NB)], sem)
        pltpu.async_copy(w_hbm.at[pl.ds(off, NB)],
                         w_buf.at[pl.ds(par * NB, NB)], sem)
        for r in range(R):
            pltpu.async_copy(
                x_hbm.at[pl.ds((row0 + r) * N + off, NB)],
                x_buf.at[pl.ds((par * R + r) * NB, NB)], sem)

    def drain():
        pltpu.make_async_copy(seg_hbm.at[pl.ds(0, NB)],
                              seg_buf.at[pl.ds(0, NB)], sem).wait()
        pltpu.make_async_copy(w_hbm.at[pl.ds(0, NB)],
                              w_buf.at[pl.ds(0, NB)], sem).wait()
        for r in range(R):
            pltpu.make_async_copy(x_hbm.at[pl.ds(0, NB)],
                                  x_buf.at[pl.ds(0, NB)], sem).wait()

    @plsc.parallel_loop(0, R * G // L, unroll=8)
    def zero_body(i):
        acc[pl.ds(i * L, L)] = zeros

    issue(0, 0)

    def block_body(blk, carry):
        par = blk & 1
        drain()

        @pl.when(blk + 1 < NUM_BLOCKS)
        def _():
            issue(blk + 1, 1 - par)

        xoff = par * (R * NB)
        soff = par * NB

        @plsc.parallel_loop(0, H, unroll=UNROLL)
        def pair_body(j):
            gs = hi_off + (j + soff)
            xv = plsc.load_gather(x_buf, [gbase + (j + xoff)])
            sv = plsc.load_gather(seg_buf, [gs])
            wv = plsc.load_gather(w_buf, [gs])
            wfv = jnp.where(sv >= HALF_G, wv, jnp.float32(1.0))
            plsc.addupdate_scatter(acc, [sbase + sv], xv * wfv)

        return carry

    lax.fori_loop(0, NUM_BLOCKS, block_body, 0)

    pltpu.sync_copy(acc, out_hbm.at[pl.ds(row0 * G, R * G)])


def kernel(x, segment_ids, W):
    mesh = plsc.VectorSubcoreMesh(core_axis_name="c", subcore_axis_name="s")
    f = pl.kernel(
        _sc_kernel,
        mesh=mesh,
        compiler_params=pltpu.CompilerParams(
            needs_layout_passes=False, use_tc_tiling_on_sc=False),
        out_type=jax.ShapeDtypeStruct((B * G,), jnp.float32),
        scratch_types=[
            pltpu.VMEM((2 * R * NB,), jnp.float32),
            pltpu.VMEM((2 * NB,), jnp.int32),
            pltpu.VMEM((2 * NB,), jnp.float32),
            pltpu.VMEM((R * G,), jnp.float32),
            pltpu.SemaphoreType.DMA,
        ],
    )
    return f(x.reshape(-1), segment_ids, W).reshape(B, G)
